# baseline (speedup 1.0000x reference)
import functools
import jax
import jax.numpy as jnp
from jax import lax
from jax.experimental import pallas as pl
from jax.experimental.pallas import tpu as pltpu
from jax.experimental.pallas import tpu_sc as plsc

_NLANE = 16
_CHUNKS = ((0, 400), (400, 400), (800, 400), (1200, 50))


def _sc_body(x_hbm, wt_hbm, pe_hbm, sp_hbm, out_hbm, wt_v, pe_v, sp_v, x_v, out_v):
    n_sp = sp_v.shape[0]
    d_model = out_v.shape[1]
    ndc = d_model // _NLANE
    nc = 2
    wid = lax.axis_index("s") * nc + lax.axis_index("c")
    b_per_w = x_hbm.shape[0] // 32

    pltpu.sync_copy(wt_hbm, wt_v)
    pltpu.sync_copy(pe_hbm, pe_v)
    pltpu.sync_copy(sp_hbm, sp_v)

    w = [[wt_v[k, pl.ds(dc * _NLANE, _NLANE)] for k in range(4)] for dc in range(ndc)]

    def make_row_body(c0):
        def row_body(r, carry):
            tok = c0 + r
            t = tok // n_sp
            s = tok - t * n_sp
            xb = 3 * tok
            xv = x_v[0, pl.ds(xb, _NLANE)]
            x0s = xv[0]
            x1s = xv[1]
            x2s = xv[2]
            m0 = x0s != x0s
            m1 = x1s != x1s
            m2 = x2s != x2s
            x0s = jnp.where(m0, 0.0, x0s)
            x1s = jnp.where(m1, 0.0, x1s)
            x2s = jnp.where(m2, 0.0, x2s)
            mfs = jnp.where(m0 | m1 | m2, 1.0, 0.0)
            x0 = jnp.full((_NLANE,), x0s, jnp.float32)
            x1 = jnp.full((_NLANE,), x1s, jnp.float32)
            x2 = jnp.full((_NLANE,), x2s, jnp.float32)
            mf = jnp.full((_NLANE,), mfs, jnp.float32)
            for dc in range(ndc):
                sl = pl.ds(dc * _NLANE, _NLANE)
                acc = pe_v[t, sl] + sp_v[s, sl]
                acc = acc + x0 * w[dc][0] + x1 * w[dc][1]
                acc = acc + x2 * w[dc][2] + mf * w[dc][3]
                out_v[r, sl] = acc
            return carry
        return row_body

    def batch_body(bi, _):
        b = wid * b_per_w + bi
        pltpu.sync_copy(x_hbm.at[b], x_v)
        for c0, csz in _CHUNKS:
            lax.fori_loop(0, csz, make_row_body(c0), 0)
            pltpu.sync_copy(out_v.at[pl.ds(0, csz)],
                            out_hbm.at[b, pl.ds(c0, csz)])
        return 0

    lax.fori_loop(0, b_per_w, batch_body, 0)


def kernel(x, W, b, space_table, nan_table, pe):
    bsize = x.shape[0]
    d_x = W.shape[1]
    d_model = W.shape[0]
    xr = x.reshape(bsize, -1, d_x)
    ntok = xr.shape[1]
    xflat = x.reshape(bsize, ntok * d_x)
    xf = jnp.pad(xflat, ((0, 0), (0, _NLANE))).reshape(bsize, 1, ntok * d_x + _NLANE)

    wt4 = jnp.concatenate([W.T, (nan_table[1] - nan_table[0])[None, :]], axis=0)
    pe_eff = pe + (b + nan_table[0])[None, :]

    mesh = plsc.VectorSubcoreMesh(core_axis_name="c", subcore_axis_name="s")
    sc_fn = functools.partial(
        pl.kernel,
        mesh=mesh,
        out_type=jax.ShapeDtypeStruct((bsize, ntok, d_model), jnp.float32),
        scratch_types=[
            pltpu.VMEM(wt4.shape, jnp.float32),
            pltpu.VMEM(pe_eff.shape, jnp.float32),
            pltpu.VMEM(space_table.shape, jnp.float32),
            pltpu.VMEM((1, ntok * d_x + _NLANE), jnp.float32),
            pltpu.VMEM((400, d_model), jnp.float32),
        ],
    )(_sc_body)
    return sc_fn(xf, wt4, pe_eff, space_table)


# speedup vs baseline: 1.9312x; 1.9312x over previous
import functools
import jax
import jax.numpy as jnp
from jax import lax
from jax.experimental import pallas as pl
from jax.experimental.pallas import tpu as pltpu
from jax.experimental.pallas import tpu_sc as plsc

_NLANE = 16
_CHUNKS = ((0, 400), (400, 400), (800, 400), (1200, 50))


def _sc_body(x_hbm, wt_hbm, pe_hbm, sp_hbm, out_hbm, wt_v, pe_v, sp_v, x_v, out_v):
    n_sp = sp_v.shape[0]
    d_model = out_v.shape[1]
    ndc = d_model // _NLANE
    nc = 2
    wid = lax.axis_index("s") * nc + lax.axis_index("c")
    b_per_w = x_hbm.shape[0] // 32

    pltpu.sync_copy(wt_hbm, wt_v)
    pltpu.sync_copy(pe_hbm, pe_v)
    pltpu.sync_copy(sp_hbm, sp_v)

    w = [[wt_v[k, pl.ds(dc * _NLANE, _NLANE)] for k in range(4)] for dc in range(ndc)]

    def make_row_body(c0):
        def row_body(r):
            tok = c0 + r
            t = tok // n_sp
            s = tok - t * n_sp
            xb = 3 * tok
            xv = x_v[0, pl.ds(xb, _NLANE)]
            x0s = xv[0]
            x1s = xv[1]
            x2s = xv[2]
            m0 = x0s != x0s
            m1 = x1s != x1s
            m2 = x2s != x2s
            x0s = jnp.where(m0, 0.0, x0s)
            x1s = jnp.where(m1, 0.0, x1s)
            x2s = jnp.where(m2, 0.0, x2s)
            mfs = jnp.where(m0 | m1 | m2, 1.0, 0.0)
            x0 = jnp.full((_NLANE,), x0s, jnp.float32)
            x1 = jnp.full((_NLANE,), x1s, jnp.float32)
            x2 = jnp.full((_NLANE,), x2s, jnp.float32)
            mf = jnp.full((_NLANE,), mfs, jnp.float32)
            for dc in range(ndc):
                sl = pl.ds(dc * _NLANE, _NLANE)
                acc = pe_v[t, sl] + sp_v[s, sl]
                acc = acc + x0 * w[dc][0] + x1 * w[dc][1]
                acc = acc + x2 * w[dc][2] + mf * w[dc][3]
                out_v[r, sl] = acc
        return row_body

    def batch_body(bi, _):
        b = wid * b_per_w + bi
        pltpu.sync_copy(x_hbm.at[b], x_v)
        for c0, csz in _CHUNKS:
            plsc.parallel_loop(0, csz, unroll=4)(make_row_body(c0))
            pltpu.sync_copy(out_v.at[pl.ds(0, csz)],
                            out_hbm.at[b, pl.ds(c0, csz)])
        return 0

    lax.fori_loop(0, b_per_w, batch_body, 0)


def kernel(x, W, b, space_table, nan_table, pe):
    bsize = x.shape[0]
    d_x = W.shape[1]
    d_model = W.shape[0]
    xr = x.reshape(bsize, -1, d_x)
    ntok = xr.shape[1]
    xflat = x.reshape(bsize, ntok * d_x)
    xf = jnp.pad(xflat, ((0, 0), (0, _NLANE))).reshape(bsize, 1, ntok * d_x + _NLANE)

    wt4 = jnp.concatenate([W.T, (nan_table[1] - nan_table[0])[None, :]], axis=0)
    pe_eff = pe + (b + nan_table[0])[None, :]

    mesh = plsc.VectorSubcoreMesh(core_axis_name="c", subcore_axis_name="s")
    sc_fn = functools.partial(
        pl.kernel,
        mesh=mesh,
        out_type=jax.ShapeDtypeStruct((bsize, ntok, d_model), jnp.float32),
        scratch_types=[
            pltpu.VMEM(wt4.shape, jnp.float32),
            pltpu.VMEM(pe_eff.shape, jnp.float32),
            pltpu.VMEM(space_table.shape, jnp.float32),
            pltpu.VMEM((1, ntok * d_x + _NLANE), jnp.float32),
            pltpu.VMEM((400, d_model), jnp.float32),
        ],
    )(_sc_body)
    return sc_fn(xf, wt4, pe_eff, space_table)
